# baseline (speedup 1.0000x reference)
import functools

import jax
import jax.numpy as jnp
from jax import lax
from jax.experimental import pallas as pl
from jax.experimental.pallas import tpu as pltpu
from jax.experimental.pallas import tpu_sc as plsc

_RW = 2048
_H = 8


def kernel(x, state):
    B, N, H = state.shape
    rows = B * N
    info = plsc.get_sparse_core_info()
    nc, ns = info.num_cores, info.num_subcores
    nw = nc * ns
    rows_per_tile = rows // nw
    n_chunks = rows_per_tile // _RW
    assert n_chunks % 2 == 0 and N % _RW == 0
    mesh = plsc.VectorSubcoreMesh(core_axis_name="c", subcore_axis_name="s")

    @functools.partial(
        pl.kernel,
        mesh=mesh,
        out_type=jax.ShapeDtypeStruct((B, N, H), state.dtype),
        scratch_types=[
            pltpu.VMEM((_RW, _H), jnp.float32),
            pltpu.VMEM((_RW, _H), jnp.float32),
            pltpu.VMEM((_RW,), jnp.float32),
            pltpu.VMEM((_RW,), jnp.float32),
            pltpu.SemaphoreType.DMA,
            pltpu.SemaphoreType.DMA,
            pltpu.SemaphoreType.DMA,
            pltpu.SemaphoreType.DMA,
            pltpu.SemaphoreType.DMA,
            pltpu.SemaphoreType.DMA,
        ],
        compiler_params=pltpu.CompilerParams(
            use_tc_tiling_on_sc=False, needs_layout_passes=False),
    )
    def push(x_hbm, s_hbm, o_hbm, ob0, ob1, xb0, xb1,
             isem0, isem1, xsem0, xsem1, osem0, osem1):
        wid = lax.axis_index("s") * nc + lax.axis_index("c")
        rbase = wid * rows_per_tile
        iota = lax.iota(jnp.int32, 16)
        zeros = iota * 0

        def in_copies(g, ob, xb, isem, xsem):
            r0 = rbase + g * _RW
            b = r0 // N
            n0 = lax.rem(r0, N)
            return (
                pltpu.make_async_copy(
                    s_hbm.at[b, pl.ds(n0, _RW), pl.ds(0, _H - 1)],
                    ob.at[:, pl.ds(1, _H - 1)], isem),
                pltpu.make_async_copy(
                    x_hbm.at[b, pl.ds(n0, _RW)], xb, xsem),
            )

        def out_copy(g, ob, osem):
            r0 = rbase + g * _RW
            return pltpu.make_async_copy(
                ob, o_hbm.at[r0 // N, pl.ds(lax.rem(r0, N), _RW), :], osem)

        def scatter_x(ob, xb):
            def body(m, carry):
                xv = xb[pl.ds(m * 16, 16)]
                plsc.store_scatter(ob, [iota + m * 16, zeros], xv)
                return carry
            lax.fori_loop(0, _RW // 16, body, 0)

        for c in in_copies(0, ob0, xb0, isem0, xsem0):
            c.start()

        def step(g2, carry):
            g = g2 * 2

            @pl.when(jnp.logical_and(g + 1 < n_chunks, g >= 1))
            def _():
                out_copy(g - 1, ob1, osem1).wait()

            @pl.when(g + 1 < n_chunks)
            def _():
                for c in in_copies(g + 1, ob1, xb1, isem1, xsem1):
                    c.start()

            for c in in_copies(g, ob0, xb0, isem0, xsem0):
                c.wait()
            scatter_x(ob0, xb0)
            out_copy(g, ob0, osem0).start()

            @pl.when(g + 2 < n_chunks)
            def _():
                out_copy(g, ob0, osem0).wait()
                for c in in_copies(g + 2, ob0, xb0, isem0, xsem0):
                    c.start()

            for c in in_copies(g + 1, ob1, xb1, isem1, xsem1):
                c.wait()
            scatter_x(ob1, xb1)
            out_copy(g + 1, ob1, osem1).start()
            return carry

        lax.fori_loop(0, n_chunks // 2, step, 0)
        out_copy(n_chunks - 2, ob0, osem0).wait()
        out_copy(n_chunks - 1, ob1, osem1).wait()

    return push(x, state)


# speedup vs baseline: 5.6445x; 5.6445x over previous
import jax
import jax.numpy as jnp
from jax import lax
from jax.experimental import pallas as pl
from jax.experimental.pallas import tpu as pltpu

_H = 8


def _push_kernel(xv_ref, s_ref, o_ref):
    s = s_ref[...]
    rolled = pltpu.roll(s, 1, axis=2)
    a1 = jnp.repeat(xv_ref[...], 8, axis=1)
    s_i = lax.broadcasted_iota(jnp.int32, s.shape, 1)
    l_i = lax.broadcasted_iota(jnp.int32, s.shape, 2)
    idx = 16 * (s_i % 8) + l_i // _H
    xr = jnp.take_along_axis(a1, idx, axis=2)
    o_ref[...] = jnp.where(l_i % _H == 0, xr, rolled)


def kernel(x, state):
    B, N, H = state.shape
    R = N * H // 128
    sv = state.reshape(B, R, 128)
    xv = x.reshape(B, N // 128, 128)
    bb, sb = 16, 64
    grid = (B // bb, R // sb)
    out = pl.pallas_call(
        _push_kernel,
        grid=grid,
        in_specs=[
            pl.BlockSpec((bb, sb // 8, 128), lambda i, j: (i, j, 0)),
            pl.BlockSpec((bb, sb, 128), lambda i, j: (i, j, 0)),
        ],
        out_specs=pl.BlockSpec((bb, sb, 128), lambda i, j: (i, j, 0)),
        out_shape=jax.ShapeDtypeStruct((B, R, 128), state.dtype),
        compiler_params=pltpu.CompilerParams(
            dimension_semantics=("parallel", "parallel"),
        ),
    )(xv, sv)
    return out.reshape(B, N, H)
